# baseline (speedup 1.0000x reference)
import jax
import jax.numpy as jnp
from jax import lax
from jax.experimental import pallas as pl
from jax.experimental.pallas import tpu as pltpu
from jax.experimental.pallas import tpu_sc as plsc

NUM_CORES = 2
NUM_SUBCORES = 16
NUM_WORKERS = NUM_CORES * NUM_SUBCORES
LANES = 16

V = 1000000
B = 16384
C = 5
D = 64
W = 2 * D
CHUNK = 16
PAIRS = CHUNK * C
GROUPS = PAIRS // LANES
B_PER_W = B // NUM_WORKERS
NCHUNKS = B_PER_W // CHUNK


def _sc_body(trow_hbm, tcol_hbm, crow_hbm, ccol_hbm, tgt_table, ctx_table,
             out_hbm, trow_v, tcol_v, crow_v, ccol_v, tgt_rows, ctx_rows,
             out_v, sem_t, sem_c):
    wid = lax.axis_index("s") * NUM_CORES + lax.axis_index("c")
    lane_iota = lax.iota(jnp.int32, LANES)

    def chunk_body(ch, carry):
        base = wid * B_PER_W + ch * CHUNK
        pltpu.sync_copy(trow_hbm.at[pl.ds(base, CHUNK)], trow_v)
        pltpu.sync_copy(tcol_hbm.at[pl.ds(base, CHUNK)], tcol_v)
        pltpu.sync_copy(crow_hbm.at[pl.ds(base * C, PAIRS)], crow_v)
        pltpu.sync_copy(ccol_hbm.at[pl.ds(base * C, PAIRS)], ccol_v)
        cp_t = pltpu.async_copy(tgt_table.at[trow_v], tgt_rows, sem_t)
        cp_c = pltpu.async_copy(ctx_table.at[crow_v], ctx_rows, sem_c)
        cp_t.wait()
        cp_c.wait()

        for g in range(GROUPS):
            rvec = jnp.int32(g * LANES) + lane_iota
            ivec = lax.shift_right_logical(rvec * jnp.int32(52429), 18)
            ccol = ccol_v[pl.ds(g * LANES, LANES)]
            tcol = plsc.load_gather(tcol_v, [ivec])
            acc = jnp.zeros((LANES,), jnp.float32)
            for d in range(D):
                cval = plsc.load_gather(ctx_rows, [rvec, ccol + d])
                tval = plsc.load_gather(tgt_rows, [ivec, tcol + d])
                acc = acc + cval * tval
            out_v[pl.ds(g * LANES, LANES)] = acc
        pltpu.sync_copy(out_v, out_hbm.at[pl.ds(base * C, PAIRS)])
        return carry

    lax.fori_loop(0, NCHUNKS, chunk_body, 0)


@jax.jit
def _run(target, context, target_table, context_table):
    tgt_idx = target.reshape(B)
    ctx_idx = context.reshape(B * C)
    trow = tgt_idx >> 1
    tcol = (tgt_idx & 1) * D
    crow = ctx_idx >> 1
    ccol = (ctx_idx & 1) * D
    scale = lax.optimization_barrier(jnp.float32(1.0))
    t2 = target_table.reshape(V // 2, W) * scale
    c2 = context_table.reshape(V // 2, W) * scale

    mesh = plsc.VectorSubcoreMesh(core_axis_name="c", subcore_axis_name="s")
    out = pl.kernel(
        _sc_body,
        out_type=jax.ShapeDtypeStruct((B * C,), jnp.float32),
        mesh=mesh,
        compiler_params=pltpu.CompilerParams(needs_layout_passes=False),
        scratch_types=[
            pltpu.VMEM((CHUNK,), jnp.int32),
            pltpu.VMEM((CHUNK,), jnp.int32),
            pltpu.VMEM((PAIRS,), jnp.int32),
            pltpu.VMEM((PAIRS,), jnp.int32),
            pltpu.VMEM((CHUNK, W), jnp.float32),
            pltpu.VMEM((PAIRS, W), jnp.float32),
            pltpu.VMEM((PAIRS,), jnp.float32),
            pltpu.SemaphoreType.DMA,
            pltpu.SemaphoreType.DMA,
        ],
    )(trow, tcol, crow, ccol, t2, c2)
    return out.reshape(B, C)


def kernel(target, context, target_table, context_table):
    return _run(target, context, target_table, context_table)


# speedup vs baseline: 1.7810x; 1.7810x over previous
import jax
import jax.numpy as jnp
from jax import lax
from jax.experimental import pallas as pl
from jax.experimental.pallas import tpu as pltpu
from jax.experimental.pallas import tpu_sc as plsc

NUM_CORES = 2
NUM_SUBCORES = 16
NW = NUM_CORES * NUM_SUBCORES
LANES = 16

V = 1000000
B = 16384
C = 5
D = 64
NCTX = B * C
NS = NCTX + B
SPW = NS // NW
GPW = SPW // LANES
NBLK = (V + 127) // 128
RING = 4
STG = 8
STG_LAG = 6
B_PER_W = B // NW
CHUNK = 16
PAIRS = CHUNK * C
NCHUNKS = B_PER_W // CHUNK
DV = D // LANES

_GDN = lax.GatherDimensionNumbers(
    offset_dims=(), collapsed_slice_dims=(0,), start_index_map=(0,))


def _perm(x, idx):
    return lax.gather(x, idx[:, None], _GDN, (1,),
                      mode=lax.GatherScatterMode.PROMISE_IN_BOUNDS)


def _scan_body(vb_h, blk_h, lo_h, hi_h, nb_h, slot_h, tT, tC, g_out,
               vb_v, blk_v, lo_v, hi_v, nb_v, slot_v, st_v, rb_v,
               sem0, sem1, sem2, sem3, sem_sc):
    wid = lax.axis_index("s") * NUM_CORES + lax.axis_index("c")
    lane_iota = lax.iota(jnp.int32, LANES)
    s0 = wid * SPW
    pltpu.sync_copy(vb_h.at[pl.ds(s0, SPW)], vb_v)
    pltpu.sync_copy(blk_h.at[pl.ds(s0, SPW)], blk_v)
    pltpu.sync_copy(lo_h.at[pl.ds(s0, SPW)], lo_v)
    pltpu.sync_copy(hi_h.at[pl.ds(s0, SPW)], hi_v)
    pltpu.sync_copy(nb_h.at[pl.ds(wid * LANES, LANES)], nb_v)
    pltpu.sync_copy(slot_h.at[pl.ds(wid * GPW, GPW)], slot_v)

    def rds(ref, i):
        base = jnp.bitwise_and(i, jnp.int32(~15))
        vv = ref[pl.ds(base, LANES)]
        return _perm(vv, lane_iota * 0 + jnp.bitwise_and(i, 15))[0]

    nblk = nb_v[pl.ds(0, LANES)][0]
    sems = [sem0, sem1, sem2, sem3]

    def issue(k, r, sem):
        @pl.when(k < nblk)
        def _():
            j = rds(blk_v, k)
            pltpu.async_copy(tT.at[:, pl.ds(j * 128, 128)], rb_v.at[2 * r], sem)
            pltpu.async_copy(tC.at[:, pl.ds(j * 128, 128)], rb_v.at[2 * r + 1], sem)

    def wait_blk(k, r, sem):
        @pl.when(k < nblk)
        def _():
            pltpu.make_async_copy(tT.at[:, pl.ds(0, 128)], rb_v.at[2 * r], sem).wait()
            pltpu.make_async_copy(tT.at[:, pl.ds(0, 128)], rb_v.at[2 * r + 1], sem).wait()

    def flush_group(fl):
        @pl.when(fl >= STG_LAG)
        def _():
            pltpu.make_async_copy(st_v.at[0], g_out.at[pl.ds(0, LANES)], sem_sc).wait()
        pltpu.async_copy(st_v.at[lax.rem(fl, jnp.int32(STG))],
                         g_out.at[slot_v.at[fl]], sem_sc)

    def extract_block(k, r, fl0):
        kc = jnp.minimum(k, SPW - 1)
        lo = rds(lo_v, kc)
        hi = rds(hi_v, kc)
        nsteps = jnp.maximum((hi - lo + LANES - 1) >> 4, 0)

        def step(t, fl):
            sv = lo + t * LANES + lane_iota
            msk = sv < hi
            svc = jnp.minimum(sv, SPW - 1)
            vb = plsc.load_gather(vb_v, [svc], mask=msk)
            i0 = 2 * r + lax.shift_right_logical(vb, 7)
            col = jnp.bitwise_and(vb, 127)
            ibuf = jnp.bitwise_and(lax.shift_right_logical(svc, 4),
                                   jnp.int32(STG - 1))
            irow = jnp.bitwise_and(svc, 15)
            for d in range(D):
                dvec = lane_iota * 0 + d
                val = plsc.load_gather(rb_v, [i0, dvec, col], mask=msk)
                plsc.store_scatter(st_v, [ibuf, irow, dvec], val, mask=msk)
            s_done = jnp.minimum(lo + (t + 1) * LANES, hi)

            def fcond(fl2):
                return (fl2 + 1) * LANES <= s_done

            def fbody(fl2):
                flush_group(fl2)
                return fl2 + 1

            return lax.while_loop(fcond, fbody, fl)

        return lax.fori_loop(0, nsteps, step, fl0)

    issue(jnp.int32(0), 0, sems[0])
    issue(jnp.int32(1), 1, sems[1])

    def super_step(s, fl):
        for r in range(RING):
            k = s * RING + r
            wait_blk(k, r, sems[r])
            fl = extract_block(k, r, fl)
            issue(k + 2, (r + 2) % RING, sems[(r + 2) % RING])
        return fl

    nsuper = (nblk + RING - 1) >> 2
    lax.fori_loop(0, nsuper, super_step, jnp.int32(0))

    for _ in range(STG_LAG):
        pltpu.make_async_copy(st_v.at[0], g_out.at[pl.ds(0, LANES)], sem_sc).wait()


def _dot_body(g_in, out_hbm, trows, crows, out_v, sem_t, sem_c):
    wid = lax.axis_index("s") * NUM_CORES + lax.axis_index("c")
    lane_iota = lax.iota(jnp.int32, LANES)
    lane_masks = [lane_iota == l for l in range(LANES)]
    xor_idx = [jnp.bitwise_xor(lane_iota, s) for s in (8, 4, 2, 1)]

    def chunk_body(ch, carry):
        base = wid * B_PER_W + ch * CHUNK
        cp_t = pltpu.async_copy(g_in.at[pl.ds(NCTX + base, CHUNK)], trows, sem_t)
        cp_c = pltpu.async_copy(g_in.at[pl.ds(base * C, PAIRS)], crows, sem_c)
        cp_t.wait()
        cp_c.wait()
        acc = [jnp.zeros((LANES,), jnp.float32) for _ in range(C)]
        for ii in range(CHUNK):
            t = [trows[ii, pl.ds(k * LANES, LANES)] for k in range(DV)]
            for c in range(C):
                rr = ii * C + c
                cv = [crows[rr, pl.ds(k * LANES, LANES)] for k in range(DV)]
                sacc = (cv[0] * t[0] + cv[1] * t[1]) + (cv[2] * t[2] + cv[3] * t[3])
                for xi in xor_idx:
                    sacc = sacc + _perm(sacc, xi)
                acc[rr // LANES] = jnp.where(lane_masks[rr % LANES], sacc,
                                             acc[rr // LANES])
        for v in range(C):
            out_v[pl.ds(v * LANES, LANES)] = acc[v]
        pltpu.sync_copy(out_v, out_hbm.at[pl.ds(base * C, PAIRS)])
        return carry

    lax.fori_loop(0, NCHUNKS, chunk_body, 0)


@jax.jit
def _run(target, context, target_table, context_table):
    i32 = jnp.int32
    tgt_idx = target.reshape(B).astype(i32)
    ctx_idx = context.reshape(NCTX).astype(i32)
    all_idx = jnp.concatenate([ctx_idx, tgt_idx])
    key = jnp.sort((lax.shift_right_logical(all_idx, 7) << 17)
                   | jnp.arange(NS, dtype=i32))
    sslot = jnp.bitwise_and(key, (1 << 17) - 1)
    sj = lax.shift_right_logical(key, 17)
    sidx = all_idx[sslot]
    vb = jnp.where(sslot < NCTX, i32(128), i32(0)) + jnp.bitwise_and(sidx, 127)
    sj2 = sj.reshape(NW, SPW)
    isnew = jnp.concatenate(
        [jnp.ones((NW, 1), jnp.bool_), sj2[:, 1:] != sj2[:, :-1]], axis=1)
    seq = jnp.cumsum(isnew.astype(i32), axis=1) - 1
    nb = seq[:, -1] + 1
    warange = jnp.broadcast_to(jnp.arange(NW, dtype=i32)[:, None], (NW, SPW))
    loc = jnp.broadcast_to(jnp.arange(SPW, dtype=i32)[None, :], (NW, SPW))
    blk = jnp.zeros((NW, SPW), i32).at[warange, seq].set(sj2, mode="drop")
    lo = jnp.full((NW, SPW), SPW, i32).at[warange, seq].min(loc, mode="drop")
    hi = jnp.zeros((NW, SPW), i32).at[warange, seq].max(loc + 1, mode="drop")
    nb16 = jnp.broadcast_to(nb[:, None], (NW, LANES))

    tT = target_table.T
    tC = context_table.T

    mesh = plsc.VectorSubcoreMesh(core_axis_name="c", subcore_axis_name="s")
    cparams = pltpu.CompilerParams(needs_layout_passes=False)
    g = pl.kernel(
        _scan_body,
        out_type=jax.ShapeDtypeStruct((NS, 2 * D), jnp.float32),
        mesh=mesh,
        compiler_params=cparams,
        scratch_types=[
            pltpu.VMEM((SPW,), i32),
            pltpu.VMEM((SPW,), i32),
            pltpu.VMEM((SPW,), i32),
            pltpu.VMEM((SPW,), i32),
            pltpu.VMEM((LANES,), i32),
            pltpu.VMEM((GPW, LANES), i32),
            pltpu.VMEM((STG, LANES, 2 * D), jnp.float32),
            pltpu.VMEM((2 * RING, 64, 128), jnp.float32),
            pltpu.SemaphoreType.DMA,
            pltpu.SemaphoreType.DMA,
            pltpu.SemaphoreType.DMA,
            pltpu.SemaphoreType.DMA,
            pltpu.SemaphoreType.DMA,
        ],
    )(vb.reshape(NS), blk.reshape(NS), lo.reshape(NS), hi.reshape(NS),
      nb16.reshape(NW * LANES), sslot.reshape(NS // LANES, LANES), tT, tC)

    out = pl.kernel(
        _dot_body,
        out_type=jax.ShapeDtypeStruct((B * C,), jnp.float32),
        mesh=mesh,
        compiler_params=cparams,
        scratch_types=[
            pltpu.VMEM((CHUNK, 2 * D), jnp.float32),
            pltpu.VMEM((PAIRS, 2 * D), jnp.float32),
            pltpu.VMEM((PAIRS,), jnp.float32),
            pltpu.SemaphoreType.DMA,
            pltpu.SemaphoreType.DMA,
        ],
    )(g)
    return out.reshape(B, C)


def kernel(target, context, target_table, context_table):
    return _run(target, context, target_table, context_table)
